# Initial kernel scaffold; baseline (speedup 1.0000x reference)
import jax
import jax.numpy as jnp
from jax import lax
from jax.experimental import pallas as pl
from jax.experimental.pallas import tpu as pltpu
from jax.experimental.pallas import tpu_sc as plsc

NUM_USERS = 25000
NUM_ITEMS = 25000
N = NUM_USERS + NUM_ITEMS
E = 800000
D = 64
N_LAYERS = 3
B = 4096

NC = 2
NS = 16
L = 16
HALF = N // NC
ACC_ROWS = 25088
RPT = ACC_ROWS // NS
ZR = 224
CH = 128
EPT = E // NS
NCH = EPT // CH
TAIL = EPT - NCH * CH
TAIL_SKIP = CH - TAIL

PB = B // (NC * NS)

_mesh = plsc.VectorSubcoreMesh(core_axis_name="c", subcore_axis_name="s")


def _mul_rows(rows_v, wm_v, n_edges):
    @pl.loop(0, n_edges, unroll=4)
    def _(i):
        w = wm_v[i]
        for q in range(D // L):
            sl = pl.ds(q * L, L)
            rows_v[i, sl] = rows_v[i, sl] * w


def _prop_body(table, src, dst, w, out,
               acc, src_v, dst_v, w_v, dstl_v, wm_v, rows_v, zb, sem):
    c = lax.axis_index("c")
    t = lax.axis_index("s")
    lo = c * HALF

    z16 = jnp.zeros((L,), jnp.float32)

    @pl.loop(0, ZR)
    def _(r):
        for q in range(D // L):
            zb[r, pl.ds(q * L, L)] = z16

    for i in range(RPT // ZR):
        pltpu.sync_copy(zb, acc.at[pl.ds(t * RPT + i * ZR, ZR)])
    plsc.subcore_barrier()

    ebase = t * EPT

    def do_chunk(base, tail):
        pltpu.sync_copy(src.at[pl.ds(base, CH)], src_v)
        pltpu.sync_copy(dst.at[pl.ds(base, CH)], dst_v)
        pltpu.sync_copy(w.at[pl.ds(base, CH)], w_v)
        gcopy = pltpu.async_copy(table.at[src_v], rows_v, sem)
        iota = lax.iota(jnp.int32, L)
        for g in range(CH // L):
            sl = pl.ds(g * L, L)
            dv = dst_v[sl]
            wv = w_v[sl]
            inr = (dv >= lo) & (dv < lo + HALF)
            if tail:
                inr = inr & (g * L + iota >= TAIL_SKIP)
            wm_v[sl] = jnp.where(inr, wv, 0.0)
            dl = jnp.minimum(jnp.maximum(dv - lo, 0), HALF - 1)
            dstl_v[sl] = dl
        gcopy.wait()
        _mul_rows(rows_v, wm_v, CH)
        pltpu.sync_copy(rows_v, acc.at[dstl_v], add=True)

    @pl.loop(0, NCH)
    def _(j):
        do_chunk(ebase + j * CH, False)

    do_chunk(ebase + EPT - CH, True)

    plsc.subcore_barrier()
    row0 = t * RPT

    @pl.when(t < NS - 1)
    def _():
        pltpu.sync_copy(acc.at[pl.ds(row0, RPT)],
                        out.at[pl.ds(lo + row0, RPT)])

    @pl.when(t == NS - 1)
    def _():
        last = HALF - (NS - 1) * RPT
        pltpu.sync_copy(acc.at[pl.ds((NS - 1) * RPT, last)],
                        out.at[pl.ds(lo + (NS - 1) * RPT, last)])


_prop = pl.kernel(
    _prop_body,
    out_type=jax.ShapeDtypeStruct((N, D), jnp.float32),
    mesh=_mesh,
    scratch_types=[
        pltpu.VMEM_SHARED((ACC_ROWS, D), jnp.float32),
        pltpu.VMEM((CH,), jnp.int32),
        pltpu.VMEM((CH,), jnp.int32),
        pltpu.VMEM((CH,), jnp.float32),
        pltpu.VMEM((CH,), jnp.int32),
        pltpu.VMEM((CH,), jnp.float32),
        pltpu.VMEM((CH, D), jnp.float32),
        pltpu.VMEM((ZR, D), jnp.float32),
        pltpu.SemaphoreType.DMA,
    ],
)


def _final_body(e0, e1, e2, e3, uidx, pidx, nidx, pos_out, neg_out,
                ui_v, pi_v, ni_v, bufs_u0, bufs_u1, bufs_u2, bufs_u3,
                bufs_p0, bufs_p1, bufs_p2, bufs_p3,
                bufs_n0, bufs_n1, bufs_n2, bufs_n3, ps_v, ns_v, sem):
    c = lax.axis_index("c")
    t = lax.axis_index("s")
    wid = c * NS + t
    base = wid * PB

    pltpu.sync_copy(uidx.at[pl.ds(base, PB)], ui_v)
    pltpu.sync_copy(pidx.at[pl.ds(base, PB)], pi_v)
    pltpu.sync_copy(nidx.at[pl.ds(base, PB)], ni_v)

    tables = (e0, e1, e2, e3)
    u_bufs = (bufs_u0, bufs_u1, bufs_u2, bufs_u3)
    p_bufs = (bufs_p0, bufs_p1, bufs_p2, bufs_p3)
    n_bufs = (bufs_n0, bufs_n1, bufs_n2, bufs_n3)
    descs = []
    for idx_v, bufs in ((ui_v, u_bufs), (pi_v, p_bufs), (ni_v, n_bufs)):
        for tb, bf in zip(tables, bufs):
            descs.append(pltpu.async_copy(tb.at[idx_v], bf, sem))
    for dsc in descs:
        dsc.wait()

    iota = lax.iota(jnp.int32, L)
    zero = jnp.zeros((L,), jnp.float32)
    for g in range(PB // L):
        ivec = g * L + iota

        def dbody(d, carry):
            accp, accn = carry
            dv = jnp.zeros((L,), jnp.int32) + d
            ud = (plsc.load_gather(bufs_u0, [ivec, dv])
                  + plsc.load_gather(bufs_u1, [ivec, dv])
                  + plsc.load_gather(bufs_u2, [ivec, dv])
                  + plsc.load_gather(bufs_u3, [ivec, dv]))
            pd = (plsc.load_gather(bufs_p0, [ivec, dv])
                  + plsc.load_gather(bufs_p1, [ivec, dv])
                  + plsc.load_gather(bufs_p2, [ivec, dv])
                  + plsc.load_gather(bufs_p3, [ivec, dv]))
            nd = (plsc.load_gather(bufs_n0, [ivec, dv])
                  + plsc.load_gather(bufs_n1, [ivec, dv])
                  + plsc.load_gather(bufs_n2, [ivec, dv])
                  + plsc.load_gather(bufs_n3, [ivec, dv]))
            return accp + ud * pd, accn + ud * nd

        accp, accn = pl.loop(0, D, init_carry=(zero, zero), unroll=4)(dbody)
        sl = pl.ds(g * L, L)
        ps_v[sl] = accp * (1.0 / ((N_LAYERS + 1) ** 2))
        ns_v[sl] = accn * (1.0 / ((N_LAYERS + 1) ** 2))

    pltpu.sync_copy(ps_v, pos_out.at[pl.ds(base, PB)])
    pltpu.sync_copy(ns_v, neg_out.at[pl.ds(base, PB)])


_final = pl.kernel(
    _final_body,
    out_type=(jax.ShapeDtypeStruct((B,), jnp.float32),
              jax.ShapeDtypeStruct((B,), jnp.float32)),
    mesh=_mesh,
    scratch_types=(
        [pltpu.VMEM((PB,), jnp.int32) for _ in range(3)]
        + [pltpu.VMEM((PB, D), jnp.float32) for _ in range(12)]
        + [pltpu.VMEM((PB,), jnp.float32) for _ in range(2)]
        + [pltpu.SemaphoreType.DMA]
    ),
)


def kernel(users, pos_items, neg_items, edge_index, edge_weight,
           user_emb, item_emb):
    table0 = jnp.concatenate([user_emb, item_emb], axis=0)
    dst = edge_index[0].astype(jnp.int32)
    src = edge_index[1].astype(jnp.int32)
    w = edge_weight

    t1 = _prop(table0, src, dst, w)
    t2 = _prop(t1, src, dst, w)
    t3 = _prop(t2, src, dst, w)

    uidx = users.astype(jnp.int32)
    pidx = pos_items.astype(jnp.int32) + NUM_USERS
    nidx = neg_items.astype(jnp.int32) + NUM_USERS
    return _final(table0, t1, t2, t3, uidx, pidx, nidx)


# speedup vs baseline: 15.9875x; 15.9875x over previous
import jax
import jax.numpy as jnp
from jax import lax
from jax.experimental import pallas as pl
from jax.experimental.pallas import tpu as pltpu
from jax.experimental.pallas import tpu_sc as plsc

NUM_USERS = 25000
NUM_ITEMS = 25000
N = NUM_USERS + NUM_ITEMS
E = 800000
D = 64
N_LAYERS = 3
B = 4096

NC = 2
NS = 16
L = 16
DH = D // NC
ACC_ROWS = 50048
RPT = ACC_ROWS // NS
CH = 320
EPT = E // NS
NCH = EPT // CH + 1
TAIL = EPT - (NCH - 1) * CH
TAIL_SKIP = CH - TAIL
NG = CH // L
NSTEADY = NCH - 3

PB = B // (NC * NS)

_mesh = plsc.VectorSubcoreMesh(core_axis_name="c", subcore_axis_name="s")
_CP = pltpu.CompilerParams(use_tc_tiling_on_sc=False, needs_layout_passes=False)


def _prop_body(table, src, dst, w, zrows, out,
               acc,
               src_v0, dst_v0, w_v0, dstl_v0, wm_v0, rows_v0,
               src_v1, dst_v1, w_v1, dstl_v1, wm_v1, rows_v1,
               isem0, isem1, gsem0, gsem1):
    c = lax.axis_index("c")
    t = lax.axis_index("s")
    cN = c * N
    bufs = ((src_v0, dst_v0, w_v0, dstl_v0, wm_v0, rows_v0, isem0, gsem0),
            (src_v1, dst_v1, w_v1, dstl_v1, wm_v1, rows_v1, isem1, gsem1))

    pltpu.sync_copy(zrows, acc.at[pl.ds(t * RPT, RPT)])
    plsc.subcore_barrier()

    ebase = t * EPT

    def issue_idx(base, b):
        src_v, dst_v, w_v = bufs[b][0], bufs[b][1], bufs[b][2]
        isem = bufs[b][6]
        pltpu.async_copy(src.at[pl.ds(base, CH)], src_v, isem)
        pltpu.async_copy(dst.at[pl.ds(base, CH)], dst_v, isem)
        pltpu.async_copy(w.at[pl.ds(base, CH)], w_v, isem)

    def wait_idx(b):
        src_v, dst_v, w_v = bufs[b][0], bufs[b][1], bufs[b][2]
        isem = bufs[b][6]
        pltpu.make_async_copy(src.at[pl.ds(0, CH)], src_v, isem).wait()
        pltpu.make_async_copy(dst.at[pl.ds(0, CH)], dst_v, isem).wait()
        pltpu.make_async_copy(w.at[pl.ds(0, CH)], w_v, isem).wait()

    def adj_src(b):
        src_v = bufs[b][0]

        @pl.loop(0, NG)
        def _(g):
            sl = pl.ds(g * L, L)
            src_v[sl] = src_v[sl] + cN

    def issue_gather(b):
        src_v, rows_v, gsem = bufs[b][0], bufs[b][5], bufs[b][7]
        pltpu.async_copy(table.at[src_v], rows_v, gsem)

    def wait_gather(b):
        src_v, rows_v, gsem = bufs[b][0], bufs[b][5], bufs[b][7]
        pltpu.make_async_copy(table.at[src_v], rows_v, gsem).wait()

    iota = lax.iota(jnp.int32, L)

    def prep(b, skip):
        dst_v, w_v, dstl_v, wm_v = (bufs[b][1], bufs[b][2],
                                    bufs[b][3], bufs[b][4])

        @pl.loop(0, NG)
        def _(g):
            sl = pl.ds(g * L, L)
            dv = dst_v[sl]
            if skip:
                dv = jnp.where(g * L + iota < skip, N, dv)
            dstl_v[sl] = dv
            wm_v[sl] = w_v[sl]

    def mul_scatter(b):
        dstl_v, wm_v, rows_v = bufs[b][3], bufs[b][4], bufs[b][5]

        @pl.loop(0, NG)
        def _(g):
            wv = wm_v[pl.ds(g * L, L)]
            for lane in range(L):
                i = g * L + lane
                wsc = wv[lane]
                for q in range(DH // L):
                    sl = pl.ds(q * L, L)
                    rows_v[i, sl] = rows_v[i, sl] * wsc
        pltpu.sync_copy(rows_v, acc.at[dstl_v], add=True)

    issue_idx(ebase, 0)
    wait_idx(0)
    adj_src(0)
    issue_gather(0)
    issue_idx(ebase + CH, 1)

    @pl.loop(0, NSTEADY, step=2)
    def _(j0):
        for bsel in range(2):
            j = j0 + bsel
            nb = bsel ^ 1
            wait_idx(nb)
            adj_src(nb)
            issue_gather(nb)
            prep(bsel, 0)
            wait_gather(bsel)
            issue_idx(ebase + (j + 2) * CH, bsel)
            mul_scatter(bsel)

    wait_idx(1)
    adj_src(1)
    issue_gather(1)
    prep(0, 0)
    wait_gather(0)
    issue_idx(ebase + EPT - CH, 0)
    mul_scatter(0)

    wait_idx(0)
    adj_src(0)
    issue_gather(0)
    prep(1, 0)
    wait_gather(1)
    mul_scatter(1)

    prep(0, TAIL_SKIP)
    wait_gather(0)
    mul_scatter(0)

    plsc.subcore_barrier()
    row0 = t * RPT

    @pl.when(t < NS - 1)
    def _():
        pltpu.sync_copy(acc.at[pl.ds(row0, RPT)],
                        out.at[pl.ds(cN + row0, RPT)])

    @pl.when(t == NS - 1)
    def _():
        last = N - (NS - 1) * RPT
        pltpu.sync_copy(acc.at[pl.ds((NS - 1) * RPT, last)],
                        out.at[pl.ds(cN + (NS - 1) * RPT, last)])


_prop = pl.kernel(
    _prop_body,
    out_type=jax.ShapeDtypeStruct((NC * N, DH), jnp.float32),
    mesh=_mesh,
    compiler_params=_CP,
    scratch_types=(
        [pltpu.VMEM_SHARED((ACC_ROWS, DH), jnp.float32)]
        + 2 * [pltpu.VMEM((CH,), jnp.int32),
               pltpu.VMEM((CH,), jnp.int32),
               pltpu.VMEM((CH,), jnp.float32),
               pltpu.VMEM((CH,), jnp.int32),
               pltpu.VMEM((CH,), jnp.float32),
               pltpu.VMEM((CH, DH), jnp.float32)]
        + 4 * [pltpu.SemaphoreType.DMA]
    ),
)


def _final_body(e0, e1, e2, e3, uidx, pidx, nidx, pos_out, neg_out,
                ui_v, pi_v, ni_v, ui2_v, pi2_v, ni2_v,
                u0l, u1l, u2l, u3l, u0h, u1h, u2h, u3h,
                p0l, p1l, p2l, p3l, p0h, p1h, p2h, p3h,
                n0l, n1l, n2l, n3l, n0h, n1h, n2h, n3h,
                ps_v, ns_v, sem):
    c = lax.axis_index("c")
    t = lax.axis_index("s")
    wid = c * NS + t
    base = wid * PB

    pltpu.sync_copy(uidx.at[pl.ds(base, PB)], ui_v)
    pltpu.sync_copy(pidx.at[pl.ds(base, PB)], pi_v)
    pltpu.sync_copy(nidx.at[pl.ds(base, PB)], ni_v)

    iota = lax.iota(jnp.int32, L)
    for iv, iv2 in ((ui_v, ui2_v), (pi_v, pi2_v), (ni_v, ni2_v)):
        @pl.loop(0, PB // L)
        def _(g):
            sl = pl.ds(g * L, L)
            iv2[sl] = iv[sl] + N

    tables = (e0, e1, e2, e3)
    gathers = (
        (ui_v, (u0l, u1l, u2l, u3l)), (ui2_v, (u0h, u1h, u2h, u3h)),
        (pi_v, (p0l, p1l, p2l, p3l)), (pi2_v, (p0h, p1h, p2h, p3h)),
        (ni_v, (n0l, n1l, n2l, n3l)), (ni2_v, (n0h, n1h, n2h, n3h)),
    )
    descs = []
    for idx_v, bl in gathers:
        for tb, bf in zip(tables, bl):
            descs.append(pltpu.async_copy(tb.at[idx_v], bf, sem))
    for dsc in descs:
        dsc.wait()

    dvecs = [jnp.full((L,), d, jnp.int32) for d in range(DH)]
    zero = jnp.zeros((L,), jnp.float32)
    u_bufs = (u0l, u1l, u2l, u3l, u0h, u1h, u2h, u3h)
    p_bufs = (p0l, p1l, p2l, p3l, p0h, p1h, p2h, p3h)
    n_bufs = (n0l, n1l, n2l, n3l, n0h, n1h, n2h, n3h)
    for g in range(PB // L):
        ivec = g * L + iota

        def dbody(d, carry):
            accp, accn = carry
            for half in range(2):
                dv = jnp.zeros((L,), jnp.int32) + d
                ud = sum(plsc.load_gather(u_bufs[4 * half + k], [ivec, dv])
                         for k in range(4))
                pd = sum(plsc.load_gather(p_bufs[4 * half + k], [ivec, dv])
                         for k in range(4))
                nd = sum(plsc.load_gather(n_bufs[4 * half + k], [ivec, dv])
                         for k in range(4))
                accp = accp + ud * pd
                accn = accn + ud * nd
            return accp, accn

        accp, accn = pl.loop(0, DH, init_carry=(zero, zero), unroll=4)(dbody)
        sl = pl.ds(g * L, L)
        ps_v[sl] = accp * (1.0 / ((N_LAYERS + 1) ** 2))
        ns_v[sl] = accn * (1.0 / ((N_LAYERS + 1) ** 2))

    pltpu.sync_copy(ps_v, pos_out.at[pl.ds(base, PB)])
    pltpu.sync_copy(ns_v, neg_out.at[pl.ds(base, PB)])


_final = pl.kernel(
    _final_body,
    out_type=(jax.ShapeDtypeStruct((B,), jnp.float32),
              jax.ShapeDtypeStruct((B,), jnp.float32)),
    mesh=_mesh,
    compiler_params=_CP,
    scratch_types=(
        [pltpu.VMEM((PB,), jnp.int32) for _ in range(6)]
        + [pltpu.VMEM((PB, DH), jnp.float32) for _ in range(24)]
        + [pltpu.VMEM((PB,), jnp.float32) for _ in range(2)]
        + [pltpu.SemaphoreType.DMA]
    ),
)


def kernel(users, pos_items, neg_items, edge_index, edge_weight,
           user_emb, item_emb):
    table0 = jnp.concatenate([user_emb, item_emb], axis=0)
    t0 = table0.reshape(N, NC, DH).transpose(1, 0, 2).reshape(NC * N, DH)
    dst = edge_index[0].astype(jnp.int32)
    src = edge_index[1].astype(jnp.int32)
    w = edge_weight

    zrows = jnp.zeros((RPT, DH), jnp.float32)
    t1 = _prop(t0, src, dst, w, zrows)
    t2 = _prop(t1, src, dst, w, zrows)
    t3 = _prop(t2, src, dst, w, zrows)

    uidx = users.astype(jnp.int32)
    pidx = pos_items.astype(jnp.int32) + NUM_USERS
    nidx = neg_items.astype(jnp.int32) + NUM_USERS
    return _final(t0, t1, t2, t3, uidx, pidx, nidx)


# speedup vs baseline: 17.0531x; 1.0667x over previous
import jax
import jax.numpy as jnp
from jax import lax
from jax.experimental import pallas as pl
from jax.experimental.pallas import tpu as pltpu
from jax.experimental.pallas import tpu_sc as plsc

NUM_USERS = 25000
NUM_ITEMS = 25000
N = NUM_USERS + NUM_ITEMS
E = 800000
D = 64
N_LAYERS = 3
B = 4096

NC = 2
NS = 16
L = 16
DH = D // NC
ACC_ROWS = 50048
RPT = ACC_ROWS // NS
CH = 320
EPT = E // NS
NCH = EPT // CH + 1
TAIL = EPT - (NCH - 1) * CH
TAIL_SKIP = CH - TAIL
NG = CH // L
NSTEADY = NCH - 3

PB = B // (NC * NS)

_mesh = plsc.VectorSubcoreMesh(core_axis_name="c", subcore_axis_name="s")
_CP = pltpu.CompilerParams(use_tc_tiling_on_sc=False, needs_layout_passes=False)


def _prop_body(table, src, dst, w, zrows, out,
               acc,
               src_v0, dst_v0, w_v0, dstl_v0, wm_v0, rows_v0,
               src_v1, dst_v1, w_v1, dstl_v1, wm_v1, rows_v1,
               isem0, isem1, gsem0, gsem1):
    c = lax.axis_index("c")
    t = lax.axis_index("s")
    cN = c * N
    bufs = ((src_v0, dst_v0, w_v0, dstl_v0, wm_v0, rows_v0, isem0, gsem0),
            (src_v1, dst_v1, w_v1, dstl_v1, wm_v1, rows_v1, isem1, gsem1))

    pltpu.sync_copy(zrows, acc.at[pl.ds(t * RPT, RPT)])
    plsc.subcore_barrier()

    ebase = t * EPT

    def issue_idx(base, b):
        src_v, dst_v, w_v = bufs[b][0], bufs[b][1], bufs[b][2]
        isem = bufs[b][6]
        pltpu.async_copy(src.at[pl.ds(base, CH)], src_v, isem)
        pltpu.async_copy(dst.at[pl.ds(base, CH)], dst_v, isem)
        pltpu.async_copy(w.at[pl.ds(base, CH)], w_v, isem)

    def wait_idx(b):
        src_v, dst_v, w_v = bufs[b][0], bufs[b][1], bufs[b][2]
        isem = bufs[b][6]
        pltpu.make_async_copy(src.at[pl.ds(0, CH)], src_v, isem).wait()
        pltpu.make_async_copy(dst.at[pl.ds(0, CH)], dst_v, isem).wait()
        pltpu.make_async_copy(w.at[pl.ds(0, CH)], w_v, isem).wait()

    def adj_src(b):
        src_v = bufs[b][0]

        @pl.loop(0, NG)
        def _(g):
            sl = pl.ds(g * L, L)
            src_v[sl] = src_v[sl] + cN

    def issue_gather(b):
        src_v, rows_v, gsem = bufs[b][0], bufs[b][5], bufs[b][7]
        pltpu.async_copy(table.at[src_v], rows_v, gsem)

    def wait_gather(b):
        src_v, rows_v, gsem = bufs[b][0], bufs[b][5], bufs[b][7]
        pltpu.make_async_copy(table.at[src_v], rows_v, gsem).wait()

    iota = lax.iota(jnp.int32, L)

    def prep(b, skip):
        dst_v, w_v, dstl_v, wm_v = (bufs[b][1], bufs[b][2],
                                    bufs[b][3], bufs[b][4])

        @pl.loop(0, NG)
        def _(g):
            sl = pl.ds(g * L, L)
            dv = dst_v[sl]
            if skip:
                dv = jnp.where(g * L + iota < skip, N, dv)
            dstl_v[sl] = dv
            wm_v[sl] = w_v[sl]

    def mul_scatter(b):
        dstl_v, wm_v, rows_v = bufs[b][3], bufs[b][4], bufs[b][5]

        @pl.loop(0, NG)
        def _(g):
            wv = wm_v[pl.ds(g * L, L)]
            for lane in range(L):
                i = g * L + lane
                wsc = wv[lane]
                for q in range(DH // L):
                    sl = pl.ds(q * L, L)
                    rows_v[i, sl] = rows_v[i, sl] * wsc
        pltpu.sync_copy(rows_v, acc.at[dstl_v], add=True)

    issue_idx(ebase, 0)
    wait_idx(0)
    adj_src(0)
    issue_gather(0)
    issue_idx(ebase + CH, 1)

    @pl.loop(0, NSTEADY, step=2)
    def _(j0):
        for bsel in range(2):
            j = j0 + bsel
            nb = bsel ^ 1
            wait_idx(nb)
            adj_src(nb)
            issue_gather(nb)
            prep(bsel, 0)
            wait_gather(bsel)
            issue_idx(ebase + (j + 2) * CH, bsel)
            mul_scatter(bsel)

    wait_idx(1)
    adj_src(1)
    issue_gather(1)
    prep(0, 0)
    wait_gather(0)
    issue_idx(ebase + EPT - CH, 0)
    mul_scatter(0)

    wait_idx(0)
    adj_src(0)
    issue_gather(0)
    prep(1, 0)
    wait_gather(1)
    mul_scatter(1)

    prep(0, TAIL_SKIP)
    wait_gather(0)
    mul_scatter(0)

    plsc.subcore_barrier()
    row0 = t * RPT

    @pl.when(t < NS - 1)
    def _():
        pltpu.sync_copy(acc.at[pl.ds(row0, RPT)],
                        out.at[pl.ds(cN + row0, RPT)])

    @pl.when(t == NS - 1)
    def _():
        last = N - (NS - 1) * RPT
        pltpu.sync_copy(acc.at[pl.ds((NS - 1) * RPT, last)],
                        out.at[pl.ds(cN + (NS - 1) * RPT, last)])


_prop = pl.kernel(
    _prop_body,
    out_type=jax.ShapeDtypeStruct((NC * N, DH), jnp.float32),
    mesh=_mesh,
    compiler_params=_CP,
    scratch_types=(
        [pltpu.VMEM_SHARED((ACC_ROWS, DH), jnp.float32)]
        + 2 * [pltpu.VMEM((CH,), jnp.int32),
               pltpu.VMEM((CH,), jnp.int32),
               pltpu.VMEM((CH,), jnp.float32),
               pltpu.VMEM((CH,), jnp.int32),
               pltpu.VMEM((CH,), jnp.float32),
               pltpu.VMEM((CH, DH), jnp.float32)]
        + 4 * [pltpu.SemaphoreType.DMA]
    ),
)


def _final_body(e0, e1, e2, e3, uidx, pidx, nidx, pos_out, neg_out,
                ui_v, pi_v, ni_v, ui2_v, pi2_v, ni2_v,
                u0l, u1l, u2l, u3l, u0h, u1h, u2h, u3h,
                p0l, p1l, p2l, p3l, p0h, p1h, p2h, p3h,
                n0l, n1l, n2l, n3l, n0h, n1h, n2h, n3h,
                ps_v, ns_v, sem):
    c = lax.axis_index("c")
    t = lax.axis_index("s")
    wid = c * NS + t
    base = wid * PB

    pltpu.sync_copy(uidx.at[pl.ds(base, PB)], ui_v)
    pltpu.sync_copy(pidx.at[pl.ds(base, PB)], pi_v)
    pltpu.sync_copy(nidx.at[pl.ds(base, PB)], ni_v)

    iota = lax.iota(jnp.int32, L)
    for iv, iv2 in ((ui_v, ui2_v), (pi_v, pi2_v), (ni_v, ni2_v)):
        @pl.loop(0, PB // L)
        def _(g):
            sl = pl.ds(g * L, L)
            iv2[sl] = iv[sl] + N

    tables = (e0, e1, e2, e3)
    gathers = (
        (ui_v, (u0l, u1l, u2l, u3l)), (ui2_v, (u0h, u1h, u2h, u3h)),
        (pi_v, (p0l, p1l, p2l, p3l)), (pi2_v, (p0h, p1h, p2h, p3h)),
        (ni_v, (n0l, n1l, n2l, n3l)), (ni2_v, (n0h, n1h, n2h, n3h)),
    )
    descs = []
    for idx_v, bl in gathers:
        for tb, bf in zip(tables, bl):
            descs.append(pltpu.async_copy(tb.at[idx_v], bf, sem))
    for dsc in descs:
        dsc.wait()

    zero = jnp.zeros((L,), jnp.float32)
    u_sets = ((u0l, u1l, u2l, u3l), (u0h, u1h, u2h, u3h))
    p_sets = ((p0l, p1l, p2l, p3l), (p0h, p1h, p2h, p3h))
    n_sets = ((n0l, n1l, n2l, n3l), (n0h, n1h, n2h, n3h))

    @pl.loop(0, PB // L)
    def _(g):
        resp = zero
        resn = zero
        for lane in range(L):
            u = g * L + lane
            accp = zero
            accn = zero
            for half in range(2):
                for q in range(DH // L):
                    sl = pl.ds(q * L, L)
                    us = sum(tb[u, sl] for tb in u_sets[half])
                    ps = sum(tb[u, sl] for tb in p_sets[half])
                    ns = sum(tb[u, sl] for tb in n_sets[half])
                    accp = accp + us * ps
                    accn = accn + us * ns
            lm = iota == lane
            resp = jnp.where(lm, jnp.sum(accp), resp)
            resn = jnp.where(lm, jnp.sum(accn), resn)
        sl = pl.ds(g * L, L)
        ps_v[sl] = resp * (1.0 / ((N_LAYERS + 1) ** 2))
        ns_v[sl] = resn * (1.0 / ((N_LAYERS + 1) ** 2))

    pltpu.sync_copy(ps_v, pos_out.at[pl.ds(base, PB)])
    pltpu.sync_copy(ns_v, neg_out.at[pl.ds(base, PB)])


_final = pl.kernel(
    _final_body,
    out_type=(jax.ShapeDtypeStruct((B,), jnp.float32),
              jax.ShapeDtypeStruct((B,), jnp.float32)),
    mesh=_mesh,
    compiler_params=_CP,
    scratch_types=(
        [pltpu.VMEM((PB,), jnp.int32) for _ in range(6)]
        + [pltpu.VMEM((PB, DH), jnp.float32) for _ in range(24)]
        + [pltpu.VMEM((PB,), jnp.float32) for _ in range(2)]
        + [pltpu.SemaphoreType.DMA]
    ),
)


def kernel(users, pos_items, neg_items, edge_index, edge_weight,
           user_emb, item_emb):
    table0 = jnp.concatenate([user_emb, item_emb], axis=0)
    t0 = table0.reshape(N, NC, DH).transpose(1, 0, 2).reshape(NC * N, DH)
    dst = edge_index[0].astype(jnp.int32)
    src = edge_index[1].astype(jnp.int32)
    w = edge_weight

    zrows = jnp.zeros((RPT, DH), jnp.float32)
    t1 = _prop(t0, src, dst, w, zrows)
    t2 = _prop(t1, src, dst, w, zrows)
    t3 = _prop(t2, src, dst, w, zrows)

    uidx = users.astype(jnp.int32)
    pidx = pos_items.astype(jnp.int32) + NUM_USERS
    nidx = neg_items.astype(jnp.int32) + NUM_USERS
    return _final(t0, t1, t2, t3, uidx, pidx, nidx)
